# baseline (speedup 1.0000x reference)
import functools

import jax
import jax.numpy as jnp
from jax import lax
from jax.experimental import pallas as pl
from jax.experimental.pallas import tpu as pltpu
from jax.experimental.pallas import tpu_sc as plsc

B = 50000
EMB = 64
DFEAT = 128

_INFO = plsc.get_sparse_core_info()
_NC = _INFO.num_cores
_NS = _INFO.num_subcores
_NW = _NC * _NS

CHUNK = 80
_NCHUNKS = B // CHUNK
_MAXC = -(-_NCHUNKS // _NW)
_FULL = _NCHUNKS - (_MAXC - 1) * _NW
_SLAB = _MAXC * CHUNK
_NID_PAD = (_NW - 1) * (_MAXC - 1) * CHUNK + _FULL * CHUNK + _SLAB - B


def _sc_body(emb2_hbm, feats_hbm, nidu2_hbm, nidi_hbm,
             pairs_hbm, rows_hbm,
             idxu_v, idxi_v, bufu_v, bufi_v, semu, semi):
    wid = lax.axis_index("s") * _NC + lax.axis_index("c")
    nchunks = jnp.where(wid < _FULL, _MAXC, _MAXC - 1)
    base = wid * ((_MAXC - 1) * CHUNK) + jnp.minimum(wid, _FULL) * CHUNK

    pltpu.sync_copy(nidu2_hbm.at[pl.ds(base, _SLAB)], idxu_v)
    pltpu.sync_copy(nidi_hbm.at[pl.ds(base, _SLAB)], idxi_v)

    for j in range(_MAXC):
        @pl.when(j < nchunks)
        def _():
            off = j * CHUNK
            cu = pltpu.async_copy(
                emb2_hbm.at[idxu_v.at[pl.ds(off, CHUNK)]], bufu_v, semu)
            ci = pltpu.async_copy(
                feats_hbm.at[idxi_v.at[pl.ds(off, CHUNK)]], bufi_v, semi)
            cu.wait()
            pltpu.sync_copy(bufu_v, pairs_hbm.at[pl.ds(base + off, CHUNK)])
            ci.wait()
            pltpu.sync_copy(bufi_v, rows_hbm.at[pl.ds(base + off, CHUNK)])


_sc_gather = functools.partial(
    pl.kernel,
    mesh=plsc.VectorSubcoreMesh(core_axis_name="c", subcore_axis_name="s"),
    out_type=[
        jax.ShapeDtypeStruct((B, 2 * EMB), jnp.float32),
        jax.ShapeDtypeStruct((B, DFEAT), jnp.float32),
    ],
    scratch_types=[
        pltpu.VMEM((_SLAB,), jnp.int32),
        pltpu.VMEM((_SLAB,), jnp.int32),
        pltpu.VMEM((CHUNK, 2 * EMB), jnp.float32),
        pltpu.VMEM((CHUNK, DFEAT), jnp.float32),
        pltpu.SemaphoreType.DMA,
        pltpu.SemaphoreType.DMA,
    ],
    compiler_params=pltpu.CompilerParams(use_tc_tiling_on_sc=True),
)(_sc_body)


def _tc_body(pairs_ref, par_ref, rows_ref, w_ref, xu_ref, xi_ref):
    pr = pairs_ref[...]
    par = par_ref[...]
    xu_ref[...] = jnp.where(par == 1, pr[:, EMB:], pr[:, :EMB])
    xi_ref[...] = jnp.dot(rows_ref[...], w_ref[...],
                          preferred_element_type=jnp.float32)


_BLK = 2000


def _tc_finish(pairs, parity, rows, w):
    return pl.pallas_call(
        _tc_body,
        grid=(B // _BLK,),
        in_specs=[
            pl.BlockSpec((_BLK, 2 * EMB), lambda i: (i, 0)),
            pl.BlockSpec((_BLK, 1), lambda i: (i, 0)),
            pl.BlockSpec((_BLK, DFEAT), lambda i: (i, 0)),
            pl.BlockSpec((DFEAT, EMB), lambda i: (0, 0)),
        ],
        out_specs=[
            pl.BlockSpec((_BLK, EMB), lambda i: (i, 0)),
            pl.BlockSpec((_BLK, EMB), lambda i: (i, 0)),
        ],
        out_shape=[
            jax.ShapeDtypeStruct((B, EMB), jnp.float32),
            jax.ShapeDtypeStruct((B, EMB), jnp.float32),
        ],
    )(pairs, parity, rows, w)


def kernel(emb_user, feats_item, W_item, nid_user, nid_item):
    nid_u = nid_user.astype(jnp.int32)
    emb2 = jnp.concatenate([emb_user[0::2], emb_user[1::2]], axis=1)
    nid_u2 = jnp.pad(nid_u >> 1, (0, _NID_PAD))
    nid_i = jnp.pad(nid_item.astype(jnp.int32), (0, _NID_PAD))
    pairs, rows = _sc_gather(emb2, feats_item, nid_u2, nid_i)
    parity = (nid_u & 1).reshape(B, 1)
    x_user, x_item = _tc_finish(pairs, parity, rows, W_item)
    return (x_user, x_item)


# speedup vs baseline: 4.7126x; 4.7126x over previous
import functools

import jax
import jax.numpy as jnp
from jax import lax
from jax.experimental import pallas as pl
from jax.experimental.pallas import tpu as pltpu
from jax.experimental.pallas import tpu_sc as plsc

B = 50000
EMB = 64
DFEAT = 128

_INFO = plsc.get_sparse_core_info()
_NC = _INFO.num_cores
_NS = _INFO.num_subcores
_NW = _NC * _NS

CHUNK = 80
_NCHUNKS = B // CHUNK
_MAXC = -(-_NCHUNKS // _NW)
_FULL = _NCHUNKS - (_MAXC - 1) * _NW
_SLAB = _MAXC * CHUNK
_NID_PAD = (_NW - 1) * (_MAXC - 1) * CHUNK + _FULL * CHUNK + _SLAB - B


def _sc_body(emb2_hbm, feats_hbm, nidu2_hbm, nidi_hbm,
             pairs_hbm, rows_hbm,
             idxu_v, idxi_v, bufu_v, bufi_v, semu, semi):
    wid = lax.axis_index("s") * _NC + lax.axis_index("c")
    nchunks = jnp.where(wid < _FULL, _MAXC, _MAXC - 1)
    base = wid * ((_MAXC - 1) * CHUNK) + jnp.minimum(wid, _FULL) * CHUNK

    pltpu.sync_copy(nidu2_hbm.at[pl.ds(base, _SLAB)], idxu_v)
    pltpu.sync_copy(nidi_hbm.at[pl.ds(base, _SLAB)], idxi_v)

    for j in range(_MAXC):
        @pl.when(j < nchunks)
        def _():
            off = j * CHUNK
            cu = pltpu.async_copy(
                emb2_hbm.at[idxu_v.at[pl.ds(off, CHUNK)]], bufu_v, semu)
            ci = pltpu.async_copy(
                feats_hbm.at[idxi_v.at[pl.ds(off, CHUNK)]], bufi_v, semi)
            cu.wait()
            pltpu.sync_copy(bufu_v, pairs_hbm.at[pl.ds(base + off, CHUNK)])
            ci.wait()
            pltpu.sync_copy(bufi_v, rows_hbm.at[pl.ds(base + off, CHUNK)])


_sc_gather = functools.partial(
    pl.kernel,
    mesh=plsc.VectorSubcoreMesh(core_axis_name="c", subcore_axis_name="s"),
    out_type=[
        jax.ShapeDtypeStruct((B, 2 * EMB), jnp.float32),
        jax.ShapeDtypeStruct((B, DFEAT), jnp.float32),
    ],
    scratch_types=[
        pltpu.VMEM((_SLAB,), jnp.int32),
        pltpu.VMEM((_SLAB,), jnp.int32),
        pltpu.VMEM((CHUNK, 2 * EMB), jnp.float32),
        pltpu.VMEM((CHUNK, DFEAT), jnp.float32),
        pltpu.SemaphoreType.DMA,
        pltpu.SemaphoreType.DMA,
    ],
    compiler_params=pltpu.CompilerParams(use_tc_tiling_on_sc=True),
)(_sc_body)


def _tc_body(pairs_ref, par_ref, rows_ref, w_ref, xu_ref, xi_ref):
    pr = pairs_ref[...]
    par = par_ref[...]
    xu_ref[...] = jnp.where(par == 1, pr[:, EMB:], pr[:, :EMB])
    xi_ref[...] = jnp.dot(rows_ref[...], w_ref[...],
                          preferred_element_type=jnp.float32)


_BLK = 2000


def _tc_finish(pairs, parity, rows, w):
    return pl.pallas_call(
        _tc_body,
        grid=(B // _BLK,),
        in_specs=[
            pl.BlockSpec((_BLK, 2 * EMB), lambda i: (i, 0)),
            pl.BlockSpec((_BLK, 1), lambda i: (i, 0)),
            pl.BlockSpec((_BLK, DFEAT), lambda i: (i, 0)),
            pl.BlockSpec((DFEAT, EMB), lambda i: (0, 0)),
        ],
        out_specs=[
            pl.BlockSpec((_BLK, EMB), lambda i: (i, 0)),
            pl.BlockSpec((_BLK, EMB), lambda i: (i, 0)),
        ],
        out_shape=[
            jax.ShapeDtypeStruct((B, EMB), jnp.float32),
            jax.ShapeDtypeStruct((B, EMB), jnp.float32),
        ],
    )(pairs, parity, rows, w)


def kernel(emb_user, feats_item, W_item, nid_user, nid_item):
    nid_u = nid_user.astype(jnp.int32)
    emb2 = emb_user.reshape(-1, 2 * EMB)
    nid_u2 = jnp.pad(nid_u >> 1, (0, _NID_PAD))
    nid_i = jnp.pad(nid_item.astype(jnp.int32), (0, _NID_PAD))
    pairs, rows = _sc_gather(emb2, feats_item, nid_u2, nid_i)
    parity = (nid_u & 1).reshape(B, 1)
    x_user, x_item = _tc_finish(pairs, parity, rows, W_item)
    return (x_user, x_item)


# speedup vs baseline: 5.1983x; 1.1031x over previous
import functools

import jax
import jax.numpy as jnp
from jax import lax
from jax.experimental import pallas as pl
from jax.experimental.pallas import tpu as pltpu
from jax.experimental.pallas import tpu_sc as plsc

B = 50000
EMB = 64
DFEAT = 128

_INFO = plsc.get_sparse_core_info()
_NC = _INFO.num_cores
_NS = _INFO.num_subcores
_NW = _NC * _NS

CHUNK = 80
_BLK = 2048
_NBLK = -(-B // _BLK)
_BP = _NBLK * _BLK
_USLAB = _BP // _NW
_UCH = _USLAB // CHUNK

_NCHUNKS = B // CHUNK
_MAXC = -(-_NCHUNKS // _NW)
_FULL = _NCHUNKS - (_MAXC - 1) * _NW
_SLAB = _MAXC * CHUNK
_NID_PAD = (_NW - 1) * (_MAXC - 1) * CHUNK + _FULL * CHUNK + _SLAB - B


def _sc_body(emb_hbm, feats_hbm, nidu_hbm, nidi_hbm,
             xu_hbm, rows_hbm,
             idxu_v, idxi_v, bufu_v, bufi_v, semu, semi):
    wid = lax.axis_index("s") * _NC + lax.axis_index("c")

    ubase = wid * _USLAB
    pltpu.sync_copy(nidu_hbm.at[pl.ds(ubase, _USLAB)], idxu_v)
    nchunks = jnp.where(wid < _FULL, _MAXC, _MAXC - 1)
    ibase = wid * ((_MAXC - 1) * CHUNK) + jnp.minimum(wid, _FULL) * CHUNK
    pltpu.sync_copy(nidi_hbm.at[pl.ds(ibase, _SLAB)], idxi_v)

    for j in range(_UCH):
        off = j * CHUNK
        cu = pltpu.async_copy(
            emb_hbm.at[idxu_v.at[pl.ds(off, CHUNK)]], bufu_v, semu)
        ci = pltpu.async_copy(
            feats_hbm.at[idxi_v.at[pl.ds(off, CHUNK)]], bufi_v, semi)
        cu.wait()
        pltpu.sync_copy(bufu_v, xu_hbm.at[pl.ds(ubase + off, CHUNK)])

        @pl.when(j < nchunks)
        def _():
            ci.wait()
            pltpu.sync_copy(bufi_v, rows_hbm.at[pl.ds(ibase + off, CHUNK)])

        @pl.when(j >= nchunks)
        def _():
            ci.wait()


_sc_gather = functools.partial(
    pl.kernel,
    mesh=plsc.VectorSubcoreMesh(core_axis_name="c", subcore_axis_name="s"),
    out_type=[
        jax.ShapeDtypeStruct((_BP, EMB), jnp.float32),
        jax.ShapeDtypeStruct((B, DFEAT), jnp.float32),
    ],
    scratch_types=[
        pltpu.VMEM((_USLAB,), jnp.int32),
        pltpu.VMEM((_SLAB,), jnp.int32),
        pltpu.VMEM((CHUNK, EMB), jnp.float32),
        pltpu.VMEM((CHUNK, DFEAT), jnp.float32),
        pltpu.SemaphoreType.DMA,
        pltpu.SemaphoreType.DMA,
    ],
    compiler_params=pltpu.CompilerParams(use_tc_tiling_on_sc=False),
)(_sc_body)


def _tc_body(xu_ref, wt_ref, rows_ref, xuT_ref, xiT_ref):
    pt = jnp.transpose(xu_ref[...])
    xuT_ref[...] = jnp.concatenate([pt[:EMB], pt[EMB:]], axis=1)
    xiT_ref[...] = lax.dot_general(
        wt_ref[...], rows_ref[...],
        dimension_numbers=(((1,), (1,)), ((), ())),
        preferred_element_type=jnp.float32)


def _tc_finish(xu_pairs, wt, rows):
    return pl.pallas_call(
        _tc_body,
        grid=(_NBLK,),
        in_specs=[
            pl.BlockSpec((_BLK // 2, 2 * EMB), lambda i: (i, 0)),
            pl.BlockSpec((EMB, DFEAT), lambda i: (0, 0)),
            pl.BlockSpec((_BLK, DFEAT), lambda i: (i, 0)),
        ],
        out_specs=[
            pl.BlockSpec((EMB, _BLK), lambda i: (0, i)),
            pl.BlockSpec((EMB, _BLK), lambda i: (0, i)),
        ],
        out_shape=[
            jax.ShapeDtypeStruct((EMB, _BP), jnp.float32),
            jax.ShapeDtypeStruct((EMB, B), jnp.float32),
        ],
    )(xu_pairs, wt, rows)


def kernel(emb_user, feats_item, W_item, nid_user, nid_item):
    nid_u = jnp.pad(nid_user.astype(jnp.int32), (0, _BP - B))
    nid_perm = nid_u.reshape(_NBLK, 2, _BLK // 2).transpose(0, 2, 1).reshape(-1)
    nid_i = jnp.pad(nid_item.astype(jnp.int32), (0, _NID_PAD))
    xu_pairs, rows = _sc_gather(emb_user, feats_item, nid_perm, nid_i)
    xu_pairs = xu_pairs.reshape(_BP // 2, 2 * EMB)
    x_userT, x_itemT = _tc_finish(xu_pairs, W_item.T, rows)
    return (x_userT[:, :B].T, x_itemT.T)


# speedup vs baseline: 5.4502x; 1.0484x over previous
import functools

import jax
import jax.numpy as jnp
from jax import lax
from jax.experimental import pallas as pl
from jax.experimental.pallas import tpu as pltpu
from jax.experimental.pallas import tpu_sc as plsc

B = 50000
EMB = 64
DFEAT = 128

_INFO = plsc.get_sparse_core_info()
_NC = _INFO.num_cores
_NS = _INFO.num_subcores
_NW = _NC * _NS

CHUNK = 80
_BLK = 2048
_NBLK = -(-B // _BLK)
_BP = _NBLK * _BLK
_USLAB = _BP // _NW
_UCH = _USLAB // CHUNK

_NCHUNKS = B // CHUNK
_MAXC = -(-_NCHUNKS // _NW)
_FULL = _NCHUNKS - (_MAXC - 1) * _NW
_SLAB = _MAXC * CHUNK
_NID_PAD = (_NW - 1) * (_MAXC - 1) * CHUNK + _FULL * CHUNK + _SLAB - B


def _sc_body(emb_hbm, feats_hbm, nidu_hbm, nidi_hbm,
             xu_hbm, rows_hbm,
             idxu_v, idxi_v, bufu_v, bufi_v, semu, semi):
    wid = lax.axis_index("s") * _NC + lax.axis_index("c")

    ubase = wid * _USLAB
    pltpu.sync_copy(nidu_hbm.at[pl.ds(ubase, _USLAB)], idxu_v)
    nchunks = jnp.where(wid < _FULL, _MAXC, _MAXC - 1)
    ibase = wid * ((_MAXC - 1) * CHUNK) + jnp.minimum(wid, _FULL) * CHUNK
    pltpu.sync_copy(nidi_hbm.at[pl.ds(ibase, _SLAB)], idxi_v)

    for j in range(_UCH):
        off = j * CHUNK
        cu = pltpu.async_copy(
            emb_hbm.at[idxu_v.at[pl.ds(off, CHUNK)]], bufu_v, semu)
        ci = pltpu.async_copy(
            feats_hbm.at[idxi_v.at[pl.ds(off, CHUNK)]], bufi_v, semi)
        cu.wait()
        pltpu.sync_copy(bufu_v, xu_hbm.at[pl.ds(ubase + off, CHUNK)])

        @pl.when(j < nchunks)
        def _():
            ci.wait()
            pltpu.sync_copy(bufi_v, rows_hbm.at[pl.ds(ibase + off, CHUNK)])

        @pl.when(j >= nchunks)
        def _():
            ci.wait()


_sc_gather = functools.partial(
    pl.kernel,
    mesh=plsc.VectorSubcoreMesh(core_axis_name="c", subcore_axis_name="s"),
    out_type=[
        jax.ShapeDtypeStruct((_BP, EMB), jnp.float32),
        jax.ShapeDtypeStruct((B, DFEAT), jnp.float32),
    ],
    scratch_types=[
        pltpu.VMEM((_USLAB,), jnp.int32),
        pltpu.VMEM((_SLAB,), jnp.int32),
        pltpu.VMEM((CHUNK, EMB), jnp.float32),
        pltpu.VMEM((CHUNK, DFEAT), jnp.float32),
        pltpu.SemaphoreType.DMA,
        pltpu.SemaphoreType.DMA,
    ],
    compiler_params=pltpu.CompilerParams(use_tc_tiling_on_sc=False),
)(_sc_body)


def _tc_body(xu_ref, wt_ref, rows_ref, xuT_ref, xiT_ref):
    pt = jnp.transpose(xu_ref[...])
    xuT_ref[...] = jnp.concatenate([pt[:EMB], pt[EMB:]], axis=1)
    xiT_ref[...] = lax.dot_general(
        wt_ref[...], rows_ref[...],
        dimension_numbers=(((1,), (1,)), ((), ())),
        preferred_element_type=jnp.float32)


def _tc_finish(xu_pairs, wt, rows):
    return pl.pallas_call(
        _tc_body,
        grid=(_NBLK,),
        in_specs=[
            pl.BlockSpec((_BLK // 2, 2 * EMB), lambda i: (i, 0)),
            pl.BlockSpec((EMB, DFEAT), lambda i: (0, 0)),
            pl.BlockSpec((_BLK, DFEAT), lambda i: (i, 0)),
        ],
        out_specs=[
            pl.BlockSpec((EMB, _BLK), lambda i: (0, i)),
            pl.BlockSpec((EMB, _BLK), lambda i: (0, i)),
        ],
        out_shape=[
            jax.ShapeDtypeStruct((EMB, B), jnp.float32),
            jax.ShapeDtypeStruct((EMB, B), jnp.float32),
        ],
    )(xu_pairs, wt, rows)


def kernel(emb_user, feats_item, W_item, nid_user, nid_item):
    nid_u = jnp.pad(nid_user.astype(jnp.int32), (0, _BP - B))
    nid_perm = nid_u.reshape(_NBLK, 2, _BLK // 2).transpose(0, 2, 1).reshape(-1)
    nid_i = jnp.pad(nid_item.astype(jnp.int32), (0, _NID_PAD))
    emb_lin = emb_user + 0.0
    xu_pairs, rows = _sc_gather(emb_lin, feats_item, nid_perm, nid_i)
    xu_pairs = xu_pairs.reshape(_BP // 2, 2 * EMB)
    x_userT, x_itemT = _tc_finish(xu_pairs, W_item.T, rows)
    return (x_userT.T, x_itemT.T)


# speedup vs baseline: 5.9813x; 1.0975x over previous
import functools

import jax
import jax.numpy as jnp
from jax import lax
from jax.experimental import pallas as pl
from jax.experimental.pallas import tpu as pltpu
from jax.experimental.pallas import tpu_sc as plsc

B = 50000
EMB = 64
DFEAT = 128

_INFO = plsc.get_sparse_core_info()
_NC = _INFO.num_cores
_NS = _INFO.num_subcores
_NW = _NC * _NS

CHUNK = 80
_BLK = 2048
_NBLK = -(-B // _BLK)
_BP = _NBLK * _BLK
_USLAB = _BP // _NW
_UCH = _USLAB // CHUNK

_NCHUNKS = B // CHUNK
_MAXC = -(-_NCHUNKS // _NW)
_FULL = _NCHUNKS - (_MAXC - 1) * _NW
_SLAB = _MAXC * CHUNK
_NID_PAD = (_NW - 1) * (_MAXC - 1) * CHUNK + _FULL * CHUNK + _SLAB - B


def _sc_user_body(emb_hbm, nidu_hbm, xu_hbm, idxu_v, bufa_v, bufb_v, sem):
    wid = lax.axis_index("s") * _NC + lax.axis_index("c")
    ubase = wid * _USLAB
    pltpu.sync_copy(nidu_hbm.at[pl.ds(ubase, _USLAB)], idxu_v)
    bufs = (bufa_v, bufb_v)
    cps = [None, None]
    for j in range(_UCH):
        off = j * CHUNK
        cps[j % 2] = pltpu.async_copy(
            emb_hbm.at[idxu_v.at[pl.ds(off, CHUNK)]], bufs[j % 2], sem)
        if j > 0:
            cps[(j - 1) % 2].wait()
            pltpu.sync_copy(bufs[(j - 1) % 2],
                            xu_hbm.at[pl.ds(ubase + (j - 1) * CHUNK, CHUNK)])
    cps[(_UCH - 1) % 2].wait()
    pltpu.sync_copy(bufs[(_UCH - 1) % 2],
                    xu_hbm.at[pl.ds(ubase + (_UCH - 1) * CHUNK, CHUNK)])


def _sc_item_body(feats_hbm, nidi_hbm, rows_hbm, idxi_v, bufa_v, bufb_v, sem):
    wid = lax.axis_index("s") * _NC + lax.axis_index("c")
    nchunks = jnp.where(wid < _FULL, _MAXC, _MAXC - 1)
    ibase = wid * ((_MAXC - 1) * CHUNK) + jnp.minimum(wid, _FULL) * CHUNK
    pltpu.sync_copy(nidi_hbm.at[pl.ds(ibase, _SLAB)], idxi_v)
    bufs = (bufa_v, bufb_v)
    cps = [None, None]
    for j in range(_MAXC):
        off = j * CHUNK
        cps[j % 2] = pltpu.async_copy(
            feats_hbm.at[idxi_v.at[pl.ds(off, CHUNK)]], bufs[j % 2], sem)
        if j > 0:
            cps[(j - 1) % 2].wait()

            @pl.when(j - 1 < nchunks)
            def _():
                pltpu.sync_copy(
                    bufs[(j - 1) % 2],
                    rows_hbm.at[pl.ds(ibase + (j - 1) * CHUNK, CHUNK)])
    cps[(_MAXC - 1) % 2].wait()

    @pl.when(_MAXC - 1 < nchunks)
    def _():
        pltpu.sync_copy(bufs[(_MAXC - 1) % 2],
                        rows_hbm.at[pl.ds(ibase + (_MAXC - 1) * CHUNK, CHUNK)])


_MESH = plsc.VectorSubcoreMesh(core_axis_name="c", subcore_axis_name="s")
_PARAMS = pltpu.CompilerParams(use_tc_tiling_on_sc=False)

_sc_user = functools.partial(
    pl.kernel,
    mesh=_MESH,
    out_type=[jax.ShapeDtypeStruct((_BP, EMB), jnp.float32)],
    scratch_types=[
        pltpu.VMEM((_USLAB,), jnp.int32),
        pltpu.VMEM((CHUNK, EMB), jnp.float32),
        pltpu.VMEM((CHUNK, EMB), jnp.float32),
        pltpu.SemaphoreType.DMA,
    ],
    compiler_params=_PARAMS,
)(_sc_user_body)

_sc_item = functools.partial(
    pl.kernel,
    mesh=_MESH,
    out_type=[jax.ShapeDtypeStruct((B, DFEAT), jnp.float32)],
    scratch_types=[
        pltpu.VMEM((_SLAB,), jnp.int32),
        pltpu.VMEM((CHUNK, DFEAT), jnp.float32),
        pltpu.VMEM((CHUNK, DFEAT), jnp.float32),
        pltpu.SemaphoreType.DMA,
    ],
    compiler_params=_PARAMS,
)(_sc_item_body)


def _tc_body(xu_ref, wt_ref, rows_ref, xuT_ref, xiT_ref):
    pt = jnp.transpose(xu_ref[...])
    xuT_ref[...] = jnp.concatenate([pt[:EMB], pt[EMB:]], axis=1)
    xiT_ref[...] = lax.dot_general(
        wt_ref[...], rows_ref[...],
        dimension_numbers=(((1,), (1,)), ((), ())),
        preferred_element_type=jnp.float32)


def _tc_finish(xu_pairs, wt, rows):
    return pl.pallas_call(
        _tc_body,
        grid=(_NBLK,),
        in_specs=[
            pl.BlockSpec((_BLK // 2, 2 * EMB), lambda i: (i, 0)),
            pl.BlockSpec((EMB, DFEAT), lambda i: (0, 0)),
            pl.BlockSpec((_BLK, DFEAT), lambda i: (i, 0)),
        ],
        out_specs=[
            pl.BlockSpec((EMB, _BLK), lambda i: (0, i)),
            pl.BlockSpec((EMB, _BLK), lambda i: (0, i)),
        ],
        out_shape=[
            jax.ShapeDtypeStruct((EMB, B), jnp.float32),
            jax.ShapeDtypeStruct((EMB, B), jnp.float32),
        ],
    )(xu_pairs, wt, rows)


def kernel(emb_user, feats_item, W_item, nid_user, nid_item):
    nid_u = jnp.pad(nid_user.astype(jnp.int32), (0, _BP - B))
    nid_perm = nid_u.reshape(_NBLK, 2, _BLK // 2).transpose(0, 2, 1).reshape(-1)
    nid_i = jnp.pad(nid_item.astype(jnp.int32), (0, _NID_PAD))
    (rows,) = _sc_item(feats_item, nid_i)
    (xu_pairs,) = _sc_user(emb_user, nid_perm)
    xu_pairs = xu_pairs.reshape(_BP // 2, 2 * EMB)
    x_userT, x_itemT = _tc_finish(xu_pairs, W_item.T, rows)
    return (x_userT.T, x_itemT.T)


# speedup vs baseline: 6.0518x; 1.0118x over previous
import functools

import jax
import jax.numpy as jnp
from jax import lax
from jax.experimental import pallas as pl
from jax.experimental.pallas import tpu as pltpu
from jax.experimental.pallas import tpu_sc as plsc

B = 50000
EMB = 64
DFEAT = 128

_INFO = plsc.get_sparse_core_info()
_NC = _INFO.num_cores
_NS = _INFO.num_subcores
_NW = _NC * _NS

CHUNK = 80
_BLK = 2048
_NBLK = -(-B // _BLK)
_BP = _NBLK * _BLK
_USLAB = _BP // _NW
_UCH = _USLAB // CHUNK

_NCHUNKS = B // CHUNK
_MAXC = -(-_NCHUNKS // _NW)
_FULL = _NCHUNKS - (_MAXC - 1) * _NW
_SLAB = _MAXC * CHUNK
_NID_PAD = (_NW - 1) * (_MAXC - 1) * CHUNK + _FULL * CHUNK + _SLAB - B


_NBUF = 4


def _sc_user_body(emb_hbm, nidu_hbm, xu_hbm, idxu_v, *bufs_sems):
    bufs, sems = bufs_sems[:_NBUF], bufs_sems[_NBUF:]
    wid = lax.axis_index("s") * _NC + lax.axis_index("c")
    ubase = wid * _USLAB
    pltpu.sync_copy(nidu_hbm.at[pl.ds(ubase, _USLAB)], idxu_v)
    cps = [None] * _NBUF
    for j in range(_UCH + _NBUF - 1):
        if j < _UCH:
            cps[j % _NBUF] = pltpu.async_copy(
                emb_hbm.at[idxu_v.at[pl.ds(j * CHUNK, CHUNK)]],
                bufs[j % _NBUF], sems[j % _NBUF])
        d = j - (_NBUF - 1)
        if 0 <= d < _UCH:
            cps[d % _NBUF].wait()
            pltpu.sync_copy(bufs[d % _NBUF],
                            xu_hbm.at[pl.ds(ubase + d * CHUNK, CHUNK)])


def _sc_item_body(feats_hbm, nidi_hbm, rows_hbm, idxi_v, *bufs_sems):
    bufs, sems = bufs_sems[:_NBUF], bufs_sems[_NBUF:]
    wid = lax.axis_index("s") * _NC + lax.axis_index("c")
    nchunks = jnp.where(wid < _FULL, _MAXC, _MAXC - 1)
    ibase = wid * ((_MAXC - 1) * CHUNK) + jnp.minimum(wid, _FULL) * CHUNK
    pltpu.sync_copy(nidi_hbm.at[pl.ds(ibase, _SLAB)], idxi_v)
    cps = [None] * _NBUF
    for j in range(_MAXC + _NBUF - 1):
        if j < _MAXC:
            cps[j % _NBUF] = pltpu.async_copy(
                feats_hbm.at[idxi_v.at[pl.ds(j * CHUNK, CHUNK)]],
                bufs[j % _NBUF], sems[j % _NBUF])
        d = j - (_NBUF - 1)
        if 0 <= d < _MAXC:
            cps[d % _NBUF].wait()

            @pl.when(d < nchunks)
            def _():
                pltpu.sync_copy(
                    bufs[d % _NBUF],
                    rows_hbm.at[pl.ds(ibase + d * CHUNK, CHUNK)])


_MESH = plsc.VectorSubcoreMesh(core_axis_name="c", subcore_axis_name="s")
_PARAMS = pltpu.CompilerParams(use_tc_tiling_on_sc=False)

_sc_user = functools.partial(
    pl.kernel,
    mesh=_MESH,
    out_type=[jax.ShapeDtypeStruct((_BP, EMB), jnp.float32)],
    scratch_types=(
        [pltpu.VMEM((_USLAB,), jnp.int32)]
        + [pltpu.VMEM((CHUNK, EMB), jnp.float32)] * _NBUF
        + [pltpu.SemaphoreType.DMA] * _NBUF
    ),
    compiler_params=_PARAMS,
)(_sc_user_body)

_sc_item = functools.partial(
    pl.kernel,
    mesh=_MESH,
    out_type=[jax.ShapeDtypeStruct((B, DFEAT), jnp.float32)],
    scratch_types=(
        [pltpu.VMEM((_SLAB,), jnp.int32)]
        + [pltpu.VMEM((CHUNK, DFEAT), jnp.float32)] * _NBUF
        + [pltpu.SemaphoreType.DMA] * _NBUF
    ),
    compiler_params=_PARAMS,
)(_sc_item_body)


def _tc_body(xu_ref, wt_ref, rows_ref, xuT_ref, xiT_ref):
    pt = jnp.transpose(xu_ref[...])
    xuT_ref[...] = jnp.concatenate([pt[:EMB], pt[EMB:]], axis=1)
    xiT_ref[...] = lax.dot_general(
        wt_ref[...], rows_ref[...],
        dimension_numbers=(((1,), (1,)), ((), ())),
        preferred_element_type=jnp.float32)


def _tc_finish(xu_pairs, wt, rows):
    return pl.pallas_call(
        _tc_body,
        grid=(_NBLK,),
        in_specs=[
            pl.BlockSpec((_BLK // 2, 2 * EMB), lambda i: (i, 0)),
            pl.BlockSpec((EMB, DFEAT), lambda i: (0, 0)),
            pl.BlockSpec((_BLK, DFEAT), lambda i: (i, 0)),
        ],
        out_specs=[
            pl.BlockSpec((EMB, _BLK), lambda i: (0, i)),
            pl.BlockSpec((EMB, _BLK), lambda i: (0, i)),
        ],
        out_shape=[
            jax.ShapeDtypeStruct((EMB, B), jnp.float32),
            jax.ShapeDtypeStruct((EMB, B), jnp.float32),
        ],
    )(xu_pairs, wt, rows)


def kernel(emb_user, feats_item, W_item, nid_user, nid_item):
    nid_u = jnp.pad(nid_user.astype(jnp.int32), (0, _BP - B))
    nid_perm = nid_u.reshape(_NBLK, 2, _BLK // 2).transpose(0, 2, 1).reshape(-1)
    nid_i = jnp.pad(nid_item.astype(jnp.int32), (0, _NID_PAD))
    (rows,) = _sc_item(feats_item, nid_i)
    (xu_pairs,) = _sc_user(emb_user, nid_perm)
    xu_pairs = xu_pairs.reshape(_BP // 2, 2 * EMB)
    x_userT, x_itemT = _tc_finish(xu_pairs, W_item.T, rows)
    return (x_userT.T, x_itemT.T)
